# Initial kernel scaffold; baseline (speedup 1.0000x reference)
import jax
import jax.numpy as jnp
from jax.experimental import pallas as pl


def kernel(X, sample_domain, W, b):
    raise NotImplementedError("write your pallas kernel here")


# speedup vs baseline: 3.5581x; 3.5581x over previous
import jax
import jax.numpy as jnp
from jax.experimental import pallas as pl
from jax.experimental.pallas import tpu as pltpu

_BI = 8


def _bcast_kernel(x_ref, w_ref, b_ref, o_ref, y_ref):
    @pl.when(pl.program_id(0) == 0)
    def _():
        y = jax.lax.dot_general(
            x_ref[...], w_ref[...], (((1,), (1,)), ((), ())),
            preferred_element_type=jnp.float32)
        y_ref[...] = jnp.maximum(y + b_ref[...], 0.0)
    o_ref[...] = jnp.broadcast_to(y_ref[...][None, :, :], o_ref.shape)


def kernel(X, sample_domain, W, b):
    n, d = X.shape
    out = pl.pallas_call(
        _bcast_kernel,
        grid=(n // _BI,),
        in_specs=[
            pl.BlockSpec((n, d), lambda i: (0, 0)),
            pl.BlockSpec((d, d), lambda i: (0, 0)),
            pl.BlockSpec((1, d), lambda i: (0, 0)),
        ],
        out_specs=pl.BlockSpec((_BI, n, d), lambda i: (i, 0, 0)),
        out_shape=jax.ShapeDtypeStruct((n, n, d), jnp.float32),
        scratch_shapes=[pltpu.VMEM((n, d), jnp.float32)],
    )(X, W, b.reshape(1, d))
    return out
